# baseline (speedup 1.0000x reference)
import functools

import jax
import jax.numpy as jnp
from jax import lax
from jax.experimental import pallas as pl
from jax.experimental.pallas import tpu as pltpu


def _flash_body(si_ref, bt_ref, scale_ref, q_ref, qrt_ref, k_ref, krt_ref,
                v_ref, o_ref, kb_s, krt_s, vb_s, *, bs, ksel, nqb, nkb,
                heads):
    h = pl.program_id(0)
    batch = h // heads
    head = h % heads
    scale2 = scale_ref[0] * 1.4426950408889634
    kb_s[...] = k_ref[0, 0].astype(jnp.bfloat16)
    krt_s[...] = krt_ref[0, 0].astype(jnp.bfloat16)
    vb_s[...] = v_ref[0, 0].astype(jnp.bfloat16)
    for qb in range(nqb):
        q = (q_ref[0, 0, qb * bs:(qb + 1) * bs, :] *
             scale2).astype(jnp.bfloat16)
        qrt = (qrt_ref[0, 0, :, qb * bs:(qb + 1) * bs] *
               scale2).astype(jnp.bfloat16)
        offs = [bt_ref[batch, si_ref[batch, head, qb, j]] * bs
                for j in range(ksel)]
        ss = []
        for off in offs:
            s = jnp.dot(q, kb_s[pl.ds(off, bs), :].T,
                        preferred_element_type=jnp.float32)
            s += lax.dot_general(qrt, krt_s[:, pl.ds(off, bs)],
                                 (((0,), (0,)), ((), ())),
                                 preferred_element_type=jnp.float32)
            ss.append(s)
        m = ss[0].max(axis=1, keepdims=True)
        for s in ss[1:]:
            m = jnp.maximum(m, s.max(axis=1, keepdims=True))
        ps = [jnp.exp2(s - m) for s in ss]
        l = ps[0].sum(axis=1, keepdims=True)
        for p in ps[1:]:
            l += p.sum(axis=1, keepdims=True)
        acc = jnp.dot(ps[0].astype(jnp.bfloat16), vb_s[pl.ds(offs[0], bs), :],
                      preferred_element_type=jnp.float32)
        for p, off in zip(ps[1:], offs[1:]):
            acc += jnp.dot(p.astype(jnp.bfloat16), vb_s[pl.ds(off, bs), :],
                           preferred_element_type=jnp.float32)
        o_ref[0, 0, qb * bs:(qb + 1) * bs, :] = acc / l


def kernel(query, key, value, sparse_indices, scale_value, block_table,
           actual_seq_lengths_query, actual_seq_lengths_kv, query_rope,
           key_rope, sparse_block_size, layout_query, layout_kv, sparse_mode,
           pre_tokens, next_tokens, attention_mode, return_softmax_lse):
    b, n, sq, d = query.shape
    dr = query_rope.shape[-1]
    skv = key.shape[2]
    nqb = sparse_indices.shape[2]
    ksel = sparse_indices.shape[3]
    bs = sq // nqb
    nkb = skv // bs
    bn = b * n

    qrt = jnp.swapaxes(query_rope, 2, 3)
    krt = jnp.swapaxes(key_rope, 2, 3)
    scale = jnp.asarray(scale_value, jnp.float32).reshape(1)

    body = functools.partial(_flash_body, bs=bs, ksel=ksel, nqb=nqb,
                             nkb=nkb, heads=n)

    def _hd(h):
        return (h // n, h % n)

    grid_spec = pltpu.PrefetchScalarGridSpec(
        num_scalar_prefetch=3,
        grid=(bn,),
        in_specs=[
            pl.BlockSpec((1, 1, sq, d), lambda h, *_: (*_hd(h), 0, 0)),
            pl.BlockSpec((1, 1, dr, sq), lambda h, *_: (*_hd(h), 0, 0)),
            pl.BlockSpec((1, 1, skv, d), lambda h, *_: (*_hd(h), 0, 0)),
            pl.BlockSpec((1, 1, dr, skv), lambda h, *_: (*_hd(h), 0, 0)),
            pl.BlockSpec((1, 1, skv, d), lambda h, *_: (*_hd(h), 0, 0)),
        ],
        out_specs=pl.BlockSpec((1, 1, sq, d), lambda h, *_: (*_hd(h), 0, 0)),
        scratch_shapes=[
            pltpu.VMEM((skv, d), jnp.bfloat16),
            pltpu.VMEM((dr, skv), jnp.bfloat16),
            pltpu.VMEM((skv, d), jnp.bfloat16),
        ],
    )
    out = pl.pallas_call(
        body,
        grid_spec=grid_spec,
        out_shape=jax.ShapeDtypeStruct((b, n, sq, d), jnp.float32),
    )(sparse_indices, block_table, scale, query, qrt, key, krt, value)
    return out


# speedup vs baseline: 1.0918x; 1.0918x over previous
import functools

import jax
import jax.numpy as jnp
from jax import lax
from jax.experimental import pallas as pl
from jax.experimental.pallas import tpu as pltpu


def _flash_body(si_ref, bt_ref, scale_ref, q_ref, qrt_ref, k_ref, krt_ref,
                v_ref, o_ref, qf_s, kf_s, vb_s, *, bs, ksel, nqb, nkb,
                heads):
    h = pl.program_id(0)
    batch = h // heads
    head = h % heads
    d = v_ref.shape[-1]
    scale2 = scale_ref[0] * 1.4426950408889634
    qf_s[:, :d] = (q_ref[0, 0] * scale2).astype(jnp.bfloat16)
    qf_s[:, d:] = (qrt_ref[0, 0] * scale2).astype(jnp.bfloat16).T
    kf_s[:, :d] = k_ref[0, 0].astype(jnp.bfloat16)
    kf_s[:, d:] = krt_ref[0, 0].astype(jnp.bfloat16).T
    vb_s[...] = v_ref[0, 0].astype(jnp.bfloat16)
    for qb in range(nqb):
        qf = qf_s[qb * bs:(qb + 1) * bs, :]
        offs = [bt_ref[batch, si_ref[batch, head, qb, j]] * bs
                for j in range(ksel)]
        kcat = jnp.concatenate([kf_s[pl.ds(off, bs), :] for off in offs],
                               axis=0)
        s = jnp.dot(qf, kcat.T, preferred_element_type=jnp.float32)
        m = jnp.max(s, axis=1, keepdims=True)
        p = jnp.exp2(s - m)
        l = jnp.sum(p, axis=1, keepdims=True)
        vcat = jnp.concatenate([vb_s[pl.ds(off, bs), :] for off in offs],
                               axis=0)
        acc = jnp.dot(p.astype(jnp.bfloat16), vcat,
                      preferred_element_type=jnp.float32)
        o_ref[0, 0, qb * bs:(qb + 1) * bs, :] = acc / l


def kernel(query, key, value, sparse_indices, scale_value, block_table,
           actual_seq_lengths_query, actual_seq_lengths_kv, query_rope,
           key_rope, sparse_block_size, layout_query, layout_kv, sparse_mode,
           pre_tokens, next_tokens, attention_mode, return_softmax_lse):
    b, n, sq, d = query.shape
    dr = query_rope.shape[-1]
    skv = key.shape[2]
    nqb = sparse_indices.shape[2]
    ksel = sparse_indices.shape[3]
    bs = sq // nqb
    nkb = skv // bs
    bn = b * n

    qrt = jnp.swapaxes(query_rope, 2, 3)
    krt = jnp.swapaxes(key_rope, 2, 3)
    scale = jnp.asarray(scale_value, jnp.float32).reshape(1)

    body = functools.partial(_flash_body, bs=bs, ksel=ksel, nqb=nqb,
                             nkb=nkb, heads=n)

    def _hd(h):
        return (h // n, h % n)

    grid_spec = pltpu.PrefetchScalarGridSpec(
        num_scalar_prefetch=3,
        grid=(bn,),
        in_specs=[
            pl.BlockSpec((1, 1, sq, d), lambda h, *_: (*_hd(h), 0, 0)),
            pl.BlockSpec((1, 1, dr, sq), lambda h, *_: (*_hd(h), 0, 0)),
            pl.BlockSpec((1, 1, skv, d), lambda h, *_: (*_hd(h), 0, 0)),
            pl.BlockSpec((1, 1, dr, skv), lambda h, *_: (*_hd(h), 0, 0)),
            pl.BlockSpec((1, 1, skv, d), lambda h, *_: (*_hd(h), 0, 0)),
        ],
        out_specs=pl.BlockSpec((1, 1, sq, d), lambda h, *_: (*_hd(h), 0, 0)),
        scratch_shapes=[
            pltpu.VMEM((sq, d + dr), jnp.bfloat16),
            pltpu.VMEM((skv, d + dr), jnp.bfloat16),
            pltpu.VMEM((skv, d), jnp.bfloat16),
        ],
    )
    out = pl.pallas_call(
        body,
        grid_spec=grid_spec,
        out_shape=jax.ShapeDtypeStruct((b, n, sq, d), jnp.float32),
    )(sparse_indices, block_table, scale, query, qrt, key, krt, value)
    return out
